# initial kernel scaffold (unmeasured)
import jax
import jax.numpy as jnp
from jax import lax
from jax.experimental import pallas as pl
from jax.experimental.pallas import tpu as pltpu

N_DEV = 16


def kernel(t, W):
    M, K = t.shape
    _, N = W.shape
    CH = M // N_DEV

    def body(t_ref, w_ref, out_ref,
             sendbuf, recvbuf, gatherbuf, wbuf, ybuf,
             rs_send_sems, rs_recv_sems, ag_send_sems, ag_recv_sems):
        my = lax.axis_index("i")

        sendbuf[...] = t_ref[...].reshape(N_DEV, CH, K).astype(jnp.bfloat16)
        wbuf[...] = w_ref[...].astype(jnp.bfloat16)

        for j in range(N_DEV):
            @pl.when(j != my)
            def _(j=j):
                pltpu.make_async_remote_copy(
                    src_ref=sendbuf.at[j],
                    dst_ref=recvbuf.at[my],
                    send_sem=rs_send_sems.at[j],
                    recv_sem=rs_recv_sems.at[my],
                    device_id=(j,),
                    device_id_type=pl.DeviceIdType.MESH,
                ).start()

        pl.store(
            recvbuf,
            (pl.dslice(my, 1), slice(None), slice(None)),
            pl.load(sendbuf, (pl.dslice(my, 1), slice(None), slice(None))),
        )

        for j in range(N_DEV):
            @pl.when(j != my)
            def _(j=j):
                pltpu.make_async_remote_copy(
                    src_ref=sendbuf.at[j],
                    dst_ref=recvbuf.at[j],
                    send_sem=rs_send_sems.at[j],
                    recv_sem=rs_recv_sems.at[j],
                    device_id=(j,),
                    device_id_type=pl.DeviceIdType.MESH,
                ).wait_recv()

        acc = jnp.sum(recvbuf[...].astype(jnp.float32), axis=0)
        y = jnp.dot(acc.astype(jnp.bfloat16), wbuf[...],
                    preferred_element_type=jnp.float32)
        ybuf[...] = y.astype(jnp.bfloat16)

        for j in range(N_DEV):
            @pl.when(j != my)
            def _(j=j):
                pltpu.make_async_remote_copy(
                    src_ref=ybuf,
                    dst_ref=gatherbuf.at[my],
                    send_sem=ag_send_sems.at[j],
                    recv_sem=ag_recv_sems.at[my],
                    device_id=(j,),
                    device_id_type=pl.DeviceIdType.MESH,
                ).start()

        pl.store(
            gatherbuf,
            (pl.dslice(my, 1), slice(None), slice(None)),
            ybuf[...][None],
        )

        for j in range(N_DEV):
            @pl.when(j != my)
            def _(j=j):
                pltpu.make_async_remote_copy(
                    src_ref=ybuf,
                    dst_ref=gatherbuf.at[j],
                    send_sem=ag_send_sems.at[j],
                    recv_sem=ag_recv_sems.at[j],
                    device_id=(j,),
                    device_id_type=pl.DeviceIdType.MESH,
                ).wait_recv()

        out_ref[...] = gatherbuf[...].reshape(M, N)

        for j in range(N_DEV):
            @pl.when(j != my)
            def _(j=j):
                pltpu.make_async_remote_copy(
                    src_ref=sendbuf.at[j],
                    dst_ref=recvbuf.at[j],
                    send_sem=rs_send_sems.at[j],
                    recv_sem=rs_recv_sems.at[j],
                    device_id=(j,),
                    device_id_type=pl.DeviceIdType.MESH,
                ).wait_send()
                pltpu.make_async_remote_copy(
                    src_ref=ybuf,
                    dst_ref=gatherbuf.at[j],
                    send_sem=ag_send_sems.at[j],
                    recv_sem=ag_recv_sems.at[j],
                    device_id=(j,),
                    device_id_type=pl.DeviceIdType.MESH,
                ).wait_send()

    return pl.pallas_call(
        body,
        out_shape=jax.ShapeDtypeStruct((M, N), jnp.bfloat16),
        in_specs=[
            pl.BlockSpec(memory_space=pltpu.VMEM),
            pl.BlockSpec(memory_space=pltpu.VMEM),
        ],
        out_specs=pl.BlockSpec(memory_space=pltpu.VMEM),
        scratch_shapes=[
            pltpu.VMEM((N_DEV, CH, K), jnp.bfloat16),
            pltpu.VMEM((N_DEV, CH, K), jnp.bfloat16),
            pltpu.VMEM((N_DEV, CH, N), jnp.bfloat16),
            pltpu.VMEM((K, N), jnp.bfloat16),
            pltpu.VMEM((CH, N), jnp.bfloat16),
            pltpu.SemaphoreType.DMA((N_DEV,)),
            pltpu.SemaphoreType.DMA((N_DEV,)),
            pltpu.SemaphoreType.DMA((N_DEV,)),
            pltpu.SemaphoreType.DMA((N_DEV,)),
        ],
    )(t, W)


# baseline (device time: 120219 ns/iter reference)
import jax
import jax.numpy as jnp
from jax import lax
from jax.experimental import pallas as pl
from jax.experimental.pallas import tpu as pltpu

N_DEV = 16


def kernel(t, W):
    M, K = t.shape
    _, N = W.shape
    CH = M // N_DEV

    def body(t_ref, w_ref, out_ref,
             sendbuf, recvbuf, gatherbuf, wbuf, ybuf,
             rs_send_sems, rs_recv_sems, ag_send_sems, ag_recv_sems):
        my = lax.axis_index("i")

        sendbuf[...] = t_ref[...].reshape(N_DEV, CH, K).astype(jnp.bfloat16)
        wbuf[...] = w_ref[...].astype(jnp.bfloat16)

        for j in range(N_DEV):
            @pl.when(j != my)
            def _(j=j):
                pltpu.make_async_remote_copy(
                    src_ref=sendbuf.at[j],
                    dst_ref=recvbuf.at[my],
                    send_sem=rs_send_sems.at[j],
                    recv_sem=rs_recv_sems.at[my],
                    device_id=(j,),
                    device_id_type=pl.DeviceIdType.MESH,
                ).start()

        recvbuf[pl.ds(my, 1), :, :] = sendbuf[pl.ds(my, 1), :, :]

        for j in range(N_DEV):
            @pl.when(j != my)
            def _(j=j):
                pltpu.make_async_remote_copy(
                    src_ref=sendbuf.at[j],
                    dst_ref=recvbuf.at[j],
                    send_sem=rs_send_sems.at[j],
                    recv_sem=rs_recv_sems.at[j],
                    device_id=(j,),
                    device_id_type=pl.DeviceIdType.MESH,
                ).wait_recv()

        acc = jnp.sum(recvbuf[...].astype(jnp.float32), axis=0)
        y = jnp.dot(acc.astype(jnp.bfloat16), wbuf[...],
                    preferred_element_type=jnp.float32)
        ybuf[...] = y.astype(jnp.bfloat16)

        for j in range(N_DEV):
            @pl.when(j != my)
            def _(j=j):
                pltpu.make_async_remote_copy(
                    src_ref=ybuf,
                    dst_ref=gatherbuf.at[my],
                    send_sem=ag_send_sems.at[j],
                    recv_sem=ag_recv_sems.at[my],
                    device_id=(j,),
                    device_id_type=pl.DeviceIdType.MESH,
                ).start()

        gatherbuf[pl.ds(my, 1), :, :] = ybuf[...][None]

        for j in range(N_DEV):
            @pl.when(j != my)
            def _(j=j):
                pltpu.make_async_remote_copy(
                    src_ref=ybuf,
                    dst_ref=gatherbuf.at[j],
                    send_sem=ag_send_sems.at[j],
                    recv_sem=ag_recv_sems.at[j],
                    device_id=(j,),
                    device_id_type=pl.DeviceIdType.MESH,
                ).wait_recv()

        out_ref[...] = gatherbuf[...].reshape(M, N)

        for j in range(N_DEV):
            @pl.when(j != my)
            def _(j=j):
                pltpu.make_async_remote_copy(
                    src_ref=sendbuf.at[j],
                    dst_ref=recvbuf.at[j],
                    send_sem=rs_send_sems.at[j],
                    recv_sem=rs_recv_sems.at[j],
                    device_id=(j,),
                    device_id_type=pl.DeviceIdType.MESH,
                ).wait_send()
                pltpu.make_async_remote_copy(
                    src_ref=ybuf,
                    dst_ref=gatherbuf.at[j],
                    send_sem=ag_send_sems.at[j],
                    recv_sem=ag_recv_sems.at[j],
                    device_id=(j,),
                    device_id_type=pl.DeviceIdType.MESH,
                ).wait_send()

    return pl.pallas_call(
        body,
        out_shape=jax.ShapeDtypeStruct((M, N), jnp.bfloat16),
        in_specs=[
            pl.BlockSpec(memory_space=pltpu.VMEM),
            pl.BlockSpec(memory_space=pltpu.VMEM),
        ],
        out_specs=pl.BlockSpec(memory_space=pltpu.VMEM),
        scratch_shapes=[
            pltpu.VMEM((N_DEV, CH, K), jnp.bfloat16),
            pltpu.VMEM((N_DEV, CH, K), jnp.bfloat16),
            pltpu.VMEM((N_DEV, CH, N), jnp.bfloat16),
            pltpu.VMEM((K, N), jnp.bfloat16),
            pltpu.VMEM((CH, N), jnp.bfloat16),
            pltpu.SemaphoreType.DMA((N_DEV,)),
            pltpu.SemaphoreType.DMA((N_DEV,)),
            pltpu.SemaphoreType.DMA((N_DEV,)),
            pltpu.SemaphoreType.DMA((N_DEV,)),
        ],
    )(t, W)


# device time: 114825 ns/iter; 1.0470x vs baseline; 1.0470x over previous
import jax
import jax.numpy as jnp
from jax import lax
from jax.experimental import pallas as pl
from jax.experimental.pallas import tpu as pltpu

N_DEV = 16


def kernel(t, W):
    M, K = t.shape
    _, N = W.shape
    CH = M // N_DEV

    def body(t_ref, w_ref, out_ref,
             sendbuf, recvbuf, wbuf,
             rs_send_sems, rs_recv_sems, ag_send_sems, ag_recv_sems):
        my = lax.axis_index("i")

        for k in range(1, N_DEV):
            tj = lax.rem(my + k, N_DEV)
            sendbuf[pl.ds(tj, 1), :, :] = (
                t_ref[pl.ds(tj * CH, CH), :].astype(jnp.bfloat16)[None]
            )
            pltpu.make_async_remote_copy(
                src_ref=sendbuf.at[tj],
                dst_ref=recvbuf.at[my],
                send_sem=rs_send_sems.at[tj],
                recv_sem=rs_recv_sems.at[my],
                device_id=(tj,),
                device_id_type=pl.DeviceIdType.MESH,
            ).start()

        wbuf[...] = w_ref[...].astype(jnp.bfloat16)
        acc = t_ref[pl.ds(my * CH, CH), :]

        for k in range(1, N_DEV):
            sj = lax.rem(my - k + N_DEV, N_DEV)
            pltpu.make_async_remote_copy(
                src_ref=sendbuf.at[sj],
                dst_ref=recvbuf.at[sj],
                send_sem=rs_send_sems.at[sj],
                recv_sem=rs_recv_sems.at[sj],
                device_id=(sj,),
                device_id_type=pl.DeviceIdType.MESH,
            ).wait_recv()
            acc = acc + recvbuf[pl.ds(sj, 1), :, :][0].astype(jnp.float32)

        y = jnp.dot(acc.astype(jnp.bfloat16), wbuf[...],
                    preferred_element_type=jnp.float32)
        out_ref[pl.ds(my * CH, CH), :] = y.astype(jnp.bfloat16)

        for k in range(1, N_DEV):
            tj = lax.rem(my + k, N_DEV)
            pltpu.make_async_remote_copy(
                src_ref=out_ref.at[pl.ds(my * CH, CH), :],
                dst_ref=out_ref.at[pl.ds(my * CH, CH), :],
                send_sem=ag_send_sems.at[tj],
                recv_sem=ag_recv_sems.at[my],
                device_id=(tj,),
                device_id_type=pl.DeviceIdType.MESH,
            ).start()

        for k in range(1, N_DEV):
            sj = lax.rem(my - k + N_DEV, N_DEV)
            pltpu.make_async_remote_copy(
                src_ref=out_ref.at[pl.ds(sj * CH, CH), :],
                dst_ref=out_ref.at[pl.ds(sj * CH, CH), :],
                send_sem=ag_send_sems.at[sj],
                recv_sem=ag_recv_sems.at[sj],
                device_id=(sj,),
                device_id_type=pl.DeviceIdType.MESH,
            ).wait_recv()

        for k in range(1, N_DEV):
            tj = lax.rem(my + k, N_DEV)
            pltpu.make_async_remote_copy(
                src_ref=sendbuf.at[tj],
                dst_ref=recvbuf.at[my],
                send_sem=rs_send_sems.at[tj],
                recv_sem=rs_recv_sems.at[my],
                device_id=(tj,),
                device_id_type=pl.DeviceIdType.MESH,
            ).wait_send()
            pltpu.make_async_remote_copy(
                src_ref=out_ref.at[pl.ds(my * CH, CH), :],
                dst_ref=out_ref.at[pl.ds(my * CH, CH), :],
                send_sem=ag_send_sems.at[tj],
                recv_sem=ag_recv_sems.at[my],
                device_id=(tj,),
                device_id_type=pl.DeviceIdType.MESH,
            ).wait_send()

    return pl.pallas_call(
        body,
        out_shape=jax.ShapeDtypeStruct((M, N), jnp.bfloat16),
        in_specs=[
            pl.BlockSpec(memory_space=pltpu.VMEM),
            pl.BlockSpec(memory_space=pltpu.VMEM),
        ],
        out_specs=pl.BlockSpec(memory_space=pltpu.VMEM),
        scratch_shapes=[
            pltpu.VMEM((N_DEV, CH, K), jnp.bfloat16),
            pltpu.VMEM((N_DEV, CH, K), jnp.bfloat16),
            pltpu.VMEM((K, N), jnp.bfloat16),
            pltpu.SemaphoreType.DMA((N_DEV,)),
            pltpu.SemaphoreType.DMA((N_DEV,)),
            pltpu.SemaphoreType.DMA((N_DEV,)),
            pltpu.SemaphoreType.DMA((N_DEV,)),
        ],
    )(t, W)


# device time: 67856 ns/iter; 1.7717x vs baseline; 1.6922x over previous
import jax
import jax.numpy as jnp
from jax import lax
from jax.experimental import pallas as pl
from jax.experimental.pallas import tpu as pltpu

N_DEV = 16
N_SCHED = 4
STEPS = 4


def kernel(t, W):
    M, K = t.shape
    _, N = W.shape
    SR = M // N_SCHED
    FR = SR // N_DEV

    step_rows = [SR >> (s + 1) for s in range(STEPS)]
    step_off = [0, 256, 384, 448]
    stage_rows_per_sched = sum(step_rows)

    def body(t_ref, w_ref, out_ref, redbuf, stage, wbuf,
             rs_send_sems, rs_recv_sems, ag_send_sems, ag_recv_sems):
        my = lax.axis_index("i")
        p = lax.rem(my, 4)
        z = my // 4

        side_x = jnp.minimum(p, 3 - p)
        side_y = p // 2
        side_zl = lax.rem(z, 2)
        side_zh = z // 2
        AX = {
            "X": (side_x, my + 1 - 2 * lax.rem(p, 2)),
            "Y": (side_y, my + 3 - 2 * p),
            "ZL": (side_zl, my + 4 * (1 - 2 * side_zl)),
            "ZH": (side_zh, my + 8 * (1 - 2 * side_zh)),
        }
        ORDERS = [
            ["X", "Y", "ZL", "ZH"],
            ["Y", "ZL", "ZH", "X"],
            ["ZL", "ZH", "X", "Y"],
            ["ZH", "X", "Y", "ZL"],
        ]

        barrier_sem = pltpu.get_barrier_semaphore()
        for ax in ("X", "Y", "ZL", "ZH"):
            pl.semaphore_signal(
                barrier_sem, inc=1,
                device_id=(AX[ax][1],), device_id_type=pl.DeviceIdType.MESH,
            )
        pl.semaphore_wait(barrier_sem, 4)

        redbuf[...] = t_ref[...].astype(jnp.bfloat16)

        send_descs = []

        def rs_copy(s, step, pt_base, rows, pt):
            off = s * stage_rows_per_sched + step_off[step]
            return pltpu.make_async_remote_copy(
                src_ref=redbuf.at[pl.ds(pt_base, rows), :],
                dst_ref=stage.at[pl.ds(off, rows), :],
                send_sem=rs_send_sems.at[4 * s + step],
                recv_sem=rs_recv_sems.at[4 * s + step],
                device_id=(pt,),
                device_id_type=pl.DeviceIdType.MESH,
            )

        bases = [jnp.int32(SR * s) for s in range(N_SCHED)]
        lens = [SR] * N_SCHED

        def rs_send(s, step):
            side, pt = AX[ORDERS[s][step]]
            h = lens[s] // 2
            pt_base = bases[s] + (1 - side) * h
            d = rs_copy(s, step, pt_base, h, pt)
            d.start()
            send_descs.append(d)

        def rs_recv_add(s, step):
            side, pt = AX[ORDERS[s][step]]
            h = lens[s] // 2
            my_base = bases[s] + side * h
            rs_copy(s, step, my_base, h, pt).wait_recv()
            off = s * stage_rows_per_sched + step_off[step]
            acc = (
                redbuf[pl.ds(my_base, h), :].astype(jnp.float32)
                + stage[pl.ds(off, h), :].astype(jnp.float32)
            )
            redbuf[pl.ds(my_base, h), :] = acc.astype(jnp.bfloat16)
            bases[s] = my_base
            lens[s] = h

        for s in range(N_SCHED):
            rs_send(s, 0)
        wbuf[...] = w_ref[...].astype(jnp.bfloat16)
        for step in range(1, STEPS):
            for s in range(N_SCHED):
                rs_recv_add(s, step - 1)
                rs_send(s, step)
        for s in range(N_SCHED):
            rs_recv_add(s, STEPS - 1)

        for s in range(N_SCHED):
            y = jnp.dot(redbuf[pl.ds(bases[s], FR), :], wbuf[...],
                        preferred_element_type=jnp.float32)
            out_ref[pl.ds(bases[s], FR), :] = y.astype(jnp.bfloat16)

        def ag_copy(s, step, src_base, dst_base, rows, pt):
            return pltpu.make_async_remote_copy(
                src_ref=out_ref.at[pl.ds(src_base, rows), :],
                dst_ref=out_ref.at[pl.ds(dst_base, rows), :],
                send_sem=ag_send_sems.at[4 * s + step],
                recv_sem=ag_recv_sems.at[4 * s + step],
                device_id=(pt,),
                device_id_type=pl.DeviceIdType.MESH,
            )

        cur_b = list(bases)
        cur_l = [FR] * N_SCHED
        for step in range(STEPS):
            sibs = []
            for s in range(N_SCHED):
                side, pt = AX[ORDERS[s][STEPS - 1 - step]]
                parent_b = cur_b[s] - side * cur_l[s]
                sib_b = parent_b + (1 - side) * cur_l[s]
                d = ag_copy(s, step, cur_b[s], cur_b[s], cur_l[s], pt)
                d.start()
                send_descs.append(d)
                sibs.append((sib_b, parent_b, pt))
            for s in range(N_SCHED):
                sib_b, parent_b, pt = sibs[s]
                ag_copy(s, step, sib_b, sib_b, cur_l[s], pt).wait_recv()
                cur_b[s] = parent_b
                cur_l[s] *= 2

        for d in send_descs:
            d.wait_send()

    return pl.pallas_call(
        body,
        out_shape=jax.ShapeDtypeStruct((M, N), jnp.bfloat16),
        in_specs=[
            pl.BlockSpec(memory_space=pltpu.VMEM),
            pl.BlockSpec(memory_space=pltpu.VMEM),
        ],
        out_specs=pl.BlockSpec(memory_space=pltpu.VMEM),
        scratch_shapes=[
            pltpu.VMEM((M, K), jnp.bfloat16),
            pltpu.VMEM((N_SCHED * stage_rows_per_sched, K), jnp.bfloat16),
            pltpu.VMEM((K, N), jnp.bfloat16),
            pltpu.SemaphoreType.DMA((16,)),
            pltpu.SemaphoreType.DMA((16,)),
            pltpu.SemaphoreType.DMA((16,)),
            pltpu.SemaphoreType.DMA((16,)),
        ],
        compiler_params=pltpu.CompilerParams(collective_id=0),
    )(t, W)
